# baseline (speedup 1.0000x reference)
import functools

import jax
import jax.numpy as jnp
from jax import lax
from jax.experimental import pallas as pl
from jax.experimental.pallas import tpu as pltpu
from jax.experimental.pallas import tpu_sc as plsc

NUM_HEADS = 32
NUM_BUCKETS = 32
Q_LEN = 2048
K_LEN = 2048
VTAB_W = 4224
THRESHOLDS = (12, 16, 23, 32, 46, 64, 91)


def _bucket16(d):
    m = jnp.abs(d)
    large = jnp.full((16,), 8, jnp.int32)
    for t in THRESHOLDS:
        large = large + jnp.where(m >= t, 1, 0)
    half = jnp.where(m < 8, m, large)
    return jnp.where(d > 0, half + 16, half)


def _sc_table_body(w_hbm, vtab_hbm, w_v, row_v, sem):
    h = lax.axis_index("s") * 2 + lax.axis_index("c")
    pltpu.sync_copy(w_hbm, w_v)
    lane = lax.iota(jnp.int32, 16)
    h_vec = jnp.zeros((16,), jnp.int32) + h

    def build(j, _):
        d = j * 16 + lane - (Q_LEN - 1)
        row_v[pl.ds(j * 16, 16)] = plsc.load_gather(
            w_v, [_bucket16(d), h_vec]
        )
        return 0

    lax.fori_loop(0, VTAB_W // 16, build, 0)
    pltpu.sync_copy(row_v, vtab_hbm.at[h, :])


def _sc_table(W):
    return functools.partial(
        pl.kernel,
        out_type=jax.ShapeDtypeStruct((NUM_HEADS, VTAB_W), jnp.float32),
        mesh=plsc.VectorSubcoreMesh(core_axis_name="c", subcore_axis_name="s"),
        scratch_types=[
            pltpu.VMEM((NUM_HEADS, NUM_BUCKETS), jnp.float32),
            pltpu.VMEM((VTAB_W,), jnp.float32),
            pltpu.SemaphoreType.DMA,
        ],
        compiler_params=pltpu.CompilerParams(
            use_tc_tiling_on_sc=False, needs_layout_passes=False
        ),
    )(_sc_table_body)(W)


WIN_W = K_LEN + 128


def _tc_expand_body(vt_ref, o_ref):
    def superblock(a, _):
        base = pl.multiple_of(8 * (240 - 16 * a), 128)
        win1 = vt_ref[0, :, pl.ds(base, WIN_W)]
        win8 = jnp.broadcast_to(win1, (8, WIN_W))
        for jm in range(16):
            sub0 = 127 - 8 * jm
            rolled = pltpu.roll(
                win8, WIN_W - sub0, 1, stride=1, stride_axis=0
            )
            o_ref[
                0, 0, pl.ds(pl.multiple_of(128 * a + 8 * jm, 8), 8), :
            ] = rolled[:, :K_LEN]
        return 0

    lax.fori_loop(0, Q_LEN // 128, superblock, 0)


def kernel(query_len, key_len, W):
    vtab = _sc_table(W).reshape(NUM_HEADS, 1, VTAB_W)
    return pl.pallas_call(
        _tc_expand_body,
        grid=(NUM_HEADS,),
        in_specs=[pl.BlockSpec((1, 1, VTAB_W), lambda i: (i, 0, 0))],
        out_specs=pl.BlockSpec(
            (1, 1, Q_LEN, K_LEN), lambda i: (0, i, 0, 0)
        ),
        out_shape=jax.ShapeDtypeStruct(
            (1, NUM_HEADS, Q_LEN, K_LEN), jnp.float32
        ),
        compiler_params=pltpu.CompilerParams(
            vmem_limit_bytes=100 * 1024 * 1024
        ),
    )(vtab)


# speedup vs baseline: 1.1389x; 1.1389x over previous
import functools

import jax
import jax.numpy as jnp
from jax import lax
from jax.experimental import pallas as pl
from jax.experimental.pallas import tpu as pltpu
from jax.experimental.pallas import tpu_sc as plsc

NUM_HEADS = 32
NUM_BUCKETS = 32
Q_LEN = 2048
K_LEN = 2048
VTAB_W = 4224
THRESHOLDS = (12, 16, 23, 32, 46, 64, 91)


def _bucket16(d):
    m = jnp.abs(d)
    large = jnp.full((16,), 8, jnp.int32)
    for t in THRESHOLDS:
        large = large + jnp.where(m >= t, 1, 0)
    half = jnp.where(m < 8, m, large)
    return jnp.where(d > 0, half + 16, half)


def _sc_table_body(w_hbm, vtab_hbm, w_v, row_v, sem):
    h = lax.axis_index("s") * 2 + lax.axis_index("c")
    pltpu.sync_copy(w_hbm, w_v)
    lane = lax.iota(jnp.int32, 16)
    h_vec = jnp.zeros((16,), jnp.int32) + h

    def build(j, _):
        d = j * 16 + lane - (Q_LEN - 1)
        row_v[pl.ds(j * 16, 16)] = plsc.load_gather(
            w_v, [_bucket16(d), h_vec]
        )
        return 0

    lax.fori_loop(0, VTAB_W // 16, build, 0)
    pltpu.sync_copy(row_v, vtab_hbm.at[h, :])


def _sc_table(W):
    return functools.partial(
        pl.kernel,
        out_type=jax.ShapeDtypeStruct((NUM_HEADS, VTAB_W), jnp.float32),
        mesh=plsc.VectorSubcoreMesh(core_axis_name="c", subcore_axis_name="s"),
        scratch_types=[
            pltpu.VMEM((NUM_HEADS, NUM_BUCKETS), jnp.float32),
            pltpu.VMEM((VTAB_W,), jnp.float32),
            pltpu.SemaphoreType.DMA,
        ],
        compiler_params=pltpu.CompilerParams(
            use_tc_tiling_on_sc=False, needs_layout_passes=False
        ),
    )(_sc_table_body)(W)


WIN_W = K_LEN + 128


BAND_LO = 1792
BAND_W = 512


def _tc_expand_body(vt_ref, o_ref):
    c15 = vt_ref[0, 0, 0]
    c31 = vt_ref[0, 0, 2 * (Q_LEN - 1)]
    vband8 = jnp.broadcast_to(vt_ref[0, :, BAND_LO : BAND_LO + BAND_W], (8, BAND_W))
    kidx = lax.broadcasted_iota(jnp.int32, (8, K_LEN), 1)

    def superblock(a, _):
        interior = jnp.logical_and(a >= 1, a <= 14)

        @pl.when(interior)
        def _fast():
            cfull = jnp.where(kidx < 128 * a, c15, c31)
            for jm in range(16):
                rows = pl.ds(pl.multiple_of(128 * a + 8 * jm, 8), 8)
                o_ref[0, 0, rows, :] = cfull
                rb = pltpu.roll(
                    vband8,
                    BAND_W - 127 + 8 * jm,
                    1,
                    stride=1,
                    stride_axis=0,
                )
                o_ref[
                    0,
                    0,
                    rows,
                    pl.ds(pl.multiple_of(128 * (a - 1), 128), 384),
                ] = rb[:, :384]

        @pl.when(jnp.logical_not(interior))
        def _edge():
            base = pl.multiple_of(8 * (240 - 16 * a), 128)
            win1 = vt_ref[0, :, pl.ds(base, WIN_W)]
            win8 = jnp.broadcast_to(win1, (8, WIN_W))
            for jm in range(16):
                sub0 = 127 - 8 * jm
                rolled = pltpu.roll(
                    win8, WIN_W - sub0, 1, stride=1, stride_axis=0
                )
                o_ref[
                    0, 0, pl.ds(pl.multiple_of(128 * a + 8 * jm, 8), 8), :
                ] = rolled[:, :K_LEN]

        return 0

    lax.fori_loop(0, Q_LEN // 128, superblock, 0)


def kernel(query_len, key_len, W):
    vtab = _sc_table(W).reshape(NUM_HEADS, 1, VTAB_W)
    return pl.pallas_call(
        _tc_expand_body,
        grid=(NUM_HEADS,),
        in_specs=[pl.BlockSpec((1, 1, VTAB_W), lambda i: (i, 0, 0))],
        out_specs=pl.BlockSpec(
            (1, 1, Q_LEN, K_LEN), lambda i: (0, i, 0, 0)
        ),
        out_shape=jax.ShapeDtypeStruct(
            (1, NUM_HEADS, Q_LEN, K_LEN), jnp.float32
        ),
        compiler_params=pltpu.CompilerParams(
            vmem_limit_bytes=100 * 1024 * 1024
        ),
    )(vtab)
